# baseline (speedup 1.0000x reference)
import functools

import jax
import jax.numpy as jnp
from jax import lax
from jax.experimental import pallas as pl
from jax.experimental.pallas import tpu as pltpu
from jax.experimental.pallas import tpu_sc as plsc

N_ATOMS = 10001
N_BONDS = 320001
ATOM_FDIM = 128
BOND_FDIM = 144
H = 128
MAX_NB = 32
DEPTH = 3
N_MOLS = 100
MOL_SIZE = 100

NC, NS = 2, 16
NW = NC * NS
IDXB = 128
AB = 4
NBATCH_A = 80
NAP = NW * NBATCH_A * AB
NBATCH_B = 80
NBG = NW * NBATCH_B * IDXB
BM = 1024
NBP = 313 * BM

_vmesh = plsc.VectorSubcoreMesh(core_axis_name="c", subcore_axis_name="s")


def _a_message(a2b_flat, wm, dst_pat):
    @functools.partial(
        pl.kernel,
        out_type=jax.ShapeDtypeStruct((NAP, H), jnp.float32),
        mesh=_vmesh,
        scratch_types=[
            pltpu.VMEM((IDXB,), jnp.int32),
            pltpu.VMEM((IDXB,), jnp.int32),
            pltpu.VMEM((IDXB, H), jnp.float32),
            pltpu.VMEM((AB, H), jnp.float32),
            pltpu.VMEM_SHARED((NS * AB, H), jnp.float32),
        ],
    )
    def k(a2b_hbm, wm_hbm, dst_hbm, out_hbm, idx_v, dst_v, rows_v, zeros_v,
          acc_sh):
        cid = lax.axis_index("c")
        sid = lax.axis_index("s")
        wid = sid * NC + cid
        pltpu.sync_copy(dst_hbm, dst_v)
        for j in range(IDXB // 16):
            sl = pl.ds(j * 16, 16)
            dst_v[sl] = dst_v[sl] + sid * AB
        for i in range(AB):
            for j in range(H // 16):
                zeros_v[i, pl.ds(j * 16, 16)] = jnp.zeros((16,), jnp.float32)

        @pl.loop(0, NBATCH_A)
        def _(b):
            base = wid * (NBATCH_A * IDXB) + b * IDXB
            pltpu.sync_copy(a2b_hbm.at[pl.ds(base, IDXB)], idx_v)
            pltpu.sync_copy(wm_hbm.at[idx_v], rows_v)
            pltpu.sync_copy(zeros_v, acc_sh.at[pl.ds(sid * AB, AB)])
            pltpu.sync_copy(rows_v, acc_sh.at[dst_v], add=True)
            row0 = wid * (NBATCH_A * AB) + b * AB
            pltpu.sync_copy(acc_sh.at[pl.ds(sid * AB, AB)],
                            out_hbm.at[pl.ds(row0, AB)])

    return k(a2b_flat, wm, dst_pat)


def _bond_gathers(b2a_p, b2revb_p, a_msg, msg):
    @functools.partial(
        pl.kernel,
        out_type=[jax.ShapeDtypeStruct((NBG, H), jnp.float32),
                  jax.ShapeDtypeStruct((NBG, H), jnp.float32)],
        mesh=_vmesh,
        scratch_types=[
            pltpu.VMEM((IDXB,), jnp.int32),
            pltpu.VMEM((IDXB,), jnp.int32),
            pltpu.VMEM((IDXB, H), jnp.float32),
            pltpu.VMEM((IDXB, H), jnp.float32),
        ],
    )
    def k(b2a_hbm, b2revb_hbm, am_hbm, msg_hbm, g1_hbm, g2_hbm,
          idx1_v, idx2_v, r1_v, r2_v):
        cid = lax.axis_index("c")
        sid = lax.axis_index("s")
        wid = sid * NC + cid

        @pl.loop(0, NBATCH_B)
        def _(b):
            base = wid * (NBATCH_B * IDXB) + b * IDXB
            pltpu.sync_copy(b2a_hbm.at[pl.ds(base, IDXB)], idx1_v)
            pltpu.sync_copy(am_hbm.at[idx1_v], r1_v)
            pltpu.sync_copy(r1_v, g1_hbm.at[pl.ds(base, IDXB)])
            pltpu.sync_copy(b2revb_hbm.at[pl.ds(base, IDXB)], idx2_v)
            pltpu.sync_copy(msg_hbm.at[idx2_v], r2_v)
            pltpu.sync_copy(r2_v, g2_hbm.at[pl.ds(base, IDXB)])

    return k(b2a_p, b2revb_p, a_msg, msg)


def _stage0(f_bonds_p, W_i, w2p):
    def body(fb_ref, wi_ref, w2_ref, inp_ref, msg_ref, wm_ref):
        h = jnp.dot(fb_ref[...], wi_ref[...],
                    preferred_element_type=jnp.float32)
        inp_ref[...] = h
        m = jnp.maximum(h, 0.0)
        msg_ref[...] = m
        wm_ref[...] = m * w2_ref[...]

    return pl.pallas_call(
        body,
        grid=(NBP // BM,),
        in_specs=[pl.BlockSpec((BM, BOND_FDIM), lambda i: (i, 0)),
                  pl.BlockSpec((BOND_FDIM, H), lambda i: (0, 0)),
                  pl.BlockSpec((BM, 1), lambda i: (i, 0))],
        out_specs=[pl.BlockSpec((BM, H), lambda i: (i, 0)),
                   pl.BlockSpec((BM, H), lambda i: (i, 0)),
                   pl.BlockSpec((BM, H), lambda i: (i, 0))],
        out_shape=[jax.ShapeDtypeStruct((NBP, H), jnp.float32)] * 3,
    )(f_bonds_p, W_i, w2p)


def _update(g1, g2, inp, W_h, w2p):
    def body(g1_ref, g2_ref, inp_ref, wh_ref, w2_ref, msg_ref, wm_ref):
        w2 = w2_ref[...]
        pre = g1_ref[...] - g2_ref[...] * w2
        h = jnp.dot(pre, wh_ref[...], preferred_element_type=jnp.float32)
        m = jnp.maximum(inp_ref[...] + h, 0.0)
        msg_ref[...] = m
        wm_ref[...] = m * w2

    return pl.pallas_call(
        body,
        grid=(NBP // BM,),
        in_specs=[pl.BlockSpec((BM, H), lambda i: (i, 0)),
                  pl.BlockSpec((BM, H), lambda i: (i, 0)),
                  pl.BlockSpec((BM, H), lambda i: (i, 0)),
                  pl.BlockSpec((H, H), lambda i: (0, 0)),
                  pl.BlockSpec((BM, 1), lambda i: (i, 0))],
        out_specs=[pl.BlockSpec((BM, H), lambda i: (i, 0)),
                   pl.BlockSpec((BM, H), lambda i: (i, 0))],
        out_shape=[jax.ShapeDtypeStruct((NBP, H), jnp.float32)] * 2,
    )(g1, g2, inp, W_h, w2p)


def _readout(f_atoms_s, a_msg_s, Wo1, Wo2, b_o2):
    RB = 400

    def body(fa_ref, am_ref, w1_ref, w2_ref, b_ref, out_ref):
        h = (jnp.dot(fa_ref[...], w1_ref[...],
                     preferred_element_type=jnp.float32)
             + jnp.dot(am_ref[...], w2_ref[...],
                       preferred_element_type=jnp.float32)
             + b_ref[...])
        out_ref[...] = jnp.maximum(h, 0.0)

    return pl.pallas_call(
        body,
        grid=(N_MOLS * MOL_SIZE // RB,),
        in_specs=[pl.BlockSpec((RB, ATOM_FDIM), lambda i: (i, 0)),
                  pl.BlockSpec((RB, H), lambda i: (i, 0)),
                  pl.BlockSpec((ATOM_FDIM, H), lambda i: (0, 0)),
                  pl.BlockSpec((H, H), lambda i: (0, 0)),
                  pl.BlockSpec((1, H), lambda i: (0, 0))],
        out_specs=pl.BlockSpec((RB, H), lambda i: (i, 0)),
        out_shape=jax.ShapeDtypeStruct((N_MOLS * MOL_SIZE, H), jnp.float32),
    )(f_atoms_s, a_msg_s, Wo1, Wo2, b_o2)


def _pool(ah3, w3, dop3):
    def body(h_ref, w_ref, d_ref, out_ref):
        x = h_ref[...]
        w = w_ref[...]
        s = jnp.sum(x * w, axis=1)
        den = jnp.sum(w, axis=1)
        out_ref[...] = s * (d_ref[..., 0] / den)

    return pl.pallas_call(
        body,
        out_shape=jax.ShapeDtypeStruct((N_MOLS, H), jnp.float32),
    )(ah3, w3, dop3)


def kernel(f_atoms, f_bonds, w_atoms, w_bonds, a2b, b2a, b2revb,
           degree_of_polym, W_i, W_h, W_o, b_o):
    i32 = jnp.int32
    a2b_flat = jnp.pad(a2b.astype(i32).reshape(-1),
                       (0, NAP * MAX_NB - N_ATOMS * MAX_NB))
    b2a_p = jnp.pad(b2a.astype(i32), (0, NBG - N_BONDS))
    b2revb_p = jnp.pad(b2revb.astype(i32), (0, NBG - N_BONDS))
    f_bonds_p = jnp.pad(f_bonds, ((0, NBP - N_BONDS), (0, 0)))
    w2p = jnp.pad(w_bonds, (0, NBP - N_BONDS))[:, None]
    dst_pat = jnp.repeat(jnp.arange(AB, dtype=i32), MAX_NB)

    inp, msg, wm = _stage0(f_bonds_p, W_i, w2p)
    for _ in range(DEPTH - 1):
        a_msg = _a_message(a2b_flat, wm, dst_pat)
        g1, g2 = _bond_gathers(b2a_p, b2revb_p, a_msg, msg)
        msg, wm = _update(g1, g2, inp, W_h, w2p)
    a_msg = _a_message(a2b_flat, wm, dst_pat)

    ah = _readout(f_atoms[1:N_ATOMS], a_msg[1:N_ATOMS],
                  W_o[:ATOM_FDIM], W_o[ATOM_FDIM:], b_o[None, :])
    mol_vecs = _pool(ah.reshape(N_MOLS, MOL_SIZE, H),
                     w_atoms[1:N_ATOMS].reshape(N_MOLS, MOL_SIZE, 1),
                     degree_of_polym[:, None, None])
    return mol_vecs


# speedup vs baseline: 1.3867x; 1.3867x over previous
import functools

import jax
import jax.numpy as jnp
from jax import lax
from jax.experimental import pallas as pl
from jax.experimental.pallas import tpu as pltpu
from jax.experimental.pallas import tpu_sc as plsc

N_ATOMS = 10001
N_BONDS = 320001
ATOM_FDIM = 128
BOND_FDIM = 144
H = 128
MAX_NB = 32
DEPTH = 3
N_MOLS = 100
MOL_SIZE = 100

NC, NS = 2, 16
NW = NC * NS
IDXB = 128
AB = 4
NBATCH_A = 80
NAP = NW * NBATCH_A * AB
NBATCH_B = 80
NBG = NW * NBATCH_B * IDXB
BM = 1024
NBP = 313 * BM

_vmesh = plsc.VectorSubcoreMesh(core_axis_name="c", subcore_axis_name="s")


def _a_message(a2b_flat, wm):
    NPAIR = NBATCH_A // 2
    APT = NBATCH_A * AB
    ZR = 32

    @functools.partial(
        pl.kernel,
        out_type=jax.ShapeDtypeStruct((NAP, H), jnp.float32),
        mesh=_vmesh,
        scratch_types=[
            pltpu.VMEM((2, IDXB), jnp.int32),
            pltpu.VMEM((2, IDXB), jnp.int32),
            pltpu.VMEM((2, IDXB, H), jnp.float32),
            pltpu.VMEM((ZR, H), jnp.float32),
            pltpu.VMEM_SHARED((NS * APT, H), jnp.float32),
            pltpu.SemaphoreType.DMA((2,)),
            pltpu.SemaphoreType.DMA((2,)),
            pltpu.SemaphoreType.DMA((2,)),
        ],
    )
    def k(a2b_hbm, wm_hbm, out_hbm, idx_v, dst_v, rows_v, zeros_v, acc_sh,
          sem_i, sem_g, sem_s):
        cid = lax.axis_index("c")
        sid = lax.axis_index("s")
        wid = sid * NC + cid
        tibase = wid * (NBATCH_A * IDXB)

        for i in range(ZR):
            for j in range(H // 16):
                zeros_v[i, pl.ds(j * 16, 16)] = jnp.zeros((16,), jnp.float32)
        for z in range(APT // ZR):
            pltpu.sync_copy(zeros_v,
                            acc_sh.at[pl.ds(sid * APT + z * ZR, ZR)])

        for p in range(2):
            off = sid * APT + p * AB
            dvp = dst_v.at[p]
            for j in range(IDXB // 16):
                dvp[pl.ds(j * 16, 16)] = (
                    jnp.full((16,), j // 2, jnp.int32) + off)

        def issue_idx(b, p):
            pltpu.async_copy(a2b_hbm.at[pl.ds(tibase + b * IDXB, IDXB)],
                             idx_v.at[p], sem_i.at[p])

        issue_idx(0, 0)
        issue_idx(1, 1)

        @pl.loop(0, NPAIR)
        def _(g):
            b0 = 2 * g
            gathers = []
            for p in range(2):
                pltpu.make_async_copy(
                    a2b_hbm.at[pl.ds(tibase, IDXB)], idx_v.at[p],
                    sem_i.at[p]).wait()
                gathers.append(pltpu.async_copy(
                    wm_hbm.at[idx_v.at[p]], rows_v.at[p], sem_g.at[p]))
            for c in gathers:
                c.wait()
            adds = []
            for p in range(2):
                adds.append(pltpu.async_copy(
                    rows_v.at[p], acc_sh.at[dst_v.at[p]], sem_s.at[p],
                    add=True))

            @pl.when(g < NPAIR - 1)
            def _():
                issue_idx(b0 + 2, 0)
                issue_idx(b0 + 3, 1)

            for c in adds:
                c.wait()
            for p in range(2):
                dvp = dst_v.at[p]
                for j in range(IDXB // 16):
                    sl = pl.ds(j * 16, 16)
                    dvp[sl] = dvp[sl] + (2 * AB)

        pltpu.sync_copy(acc_sh.at[pl.ds(sid * APT, APT)],
                        out_hbm.at[pl.ds(wid * APT, APT)])

    return k(a2b_flat, wm)


def _bond_gathers(b2a_p, b2revb_p, a_msg, msg):
    NPAIR = NBATCH_B // 2

    @functools.partial(
        pl.kernel,
        out_type=[jax.ShapeDtypeStruct((NBG, H), jnp.float32),
                  jax.ShapeDtypeStruct((NBG, H), jnp.float32)],
        mesh=_vmesh,
        scratch_types=[
            pltpu.VMEM((2, IDXB), jnp.int32),
            pltpu.VMEM((2, IDXB), jnp.int32),
            pltpu.VMEM((2, IDXB, H), jnp.float32),
            pltpu.VMEM((2, IDXB, H), jnp.float32),
            pltpu.SemaphoreType.DMA((2,)),
            pltpu.SemaphoreType.DMA((2,)),
            pltpu.SemaphoreType.DMA((2,)),
        ],
    )
    def k(b2a_hbm, b2revb_hbm, am_hbm, msg_hbm, g1_hbm, g2_hbm,
          idx1_v, idx2_v, r1_v, r2_v, sem_i, sem_g, sem_w):
        cid = lax.axis_index("c")
        sid = lax.axis_index("s")
        wid = sid * NC + cid
        tbase = wid * (NBATCH_B * IDXB)

        def issue_idx(b, p):
            base = tbase + b * IDXB
            pltpu.async_copy(b2a_hbm.at[pl.ds(base, IDXB)], idx1_v.at[p],
                             sem_i.at[p])
            pltpu.async_copy(b2revb_hbm.at[pl.ds(base, IDXB)], idx2_v.at[p],
                             sem_i.at[p])

        issue_idx(0, 0)
        issue_idx(1, 1)

        @pl.loop(0, NPAIR)
        def _(g):
            b0 = 2 * g

            @pl.when(g > 0)
            def _():
                for p in range(2):
                    pltpu.make_async_copy(r1_v.at[p],
                                          g1_hbm.at[pl.ds(0, IDXB)],
                                          sem_w.at[p]).wait()
                    pltpu.make_async_copy(r2_v.at[p],
                                          g2_hbm.at[pl.ds(0, IDXB)],
                                          sem_w.at[p]).wait()

            gathers = []
            for p in range(2):
                pltpu.make_async_copy(b2a_hbm.at[pl.ds(tbase, IDXB)],
                                      idx1_v.at[p], sem_i.at[p]).wait()
                pltpu.make_async_copy(b2revb_hbm.at[pl.ds(tbase, IDXB)],
                                      idx2_v.at[p], sem_i.at[p]).wait()
                gathers.append(pltpu.async_copy(
                    am_hbm.at[idx1_v.at[p]], r1_v.at[p], sem_g.at[p]))
                gathers.append(pltpu.async_copy(
                    msg_hbm.at[idx2_v.at[p]], r2_v.at[p], sem_g.at[p]))
            for c in gathers:
                c.wait()
            for p in range(2):
                base = tbase + (b0 + p) * IDXB
                pltpu.async_copy(r1_v.at[p], g1_hbm.at[pl.ds(base, IDXB)],
                                 sem_w.at[p])
                pltpu.async_copy(r2_v.at[p], g2_hbm.at[pl.ds(base, IDXB)],
                                 sem_w.at[p])

            @pl.when(g < NPAIR - 1)
            def _():
                issue_idx(b0 + 2, 0)
                issue_idx(b0 + 3, 1)

        for p in range(2):
            pltpu.make_async_copy(r1_v.at[p], g1_hbm.at[pl.ds(0, IDXB)],
                                  sem_w.at[p]).wait()
            pltpu.make_async_copy(r2_v.at[p], g2_hbm.at[pl.ds(0, IDXB)],
                                  sem_w.at[p]).wait()

    return k(b2a_p, b2revb_p, a_msg, msg)


def _stage0(f_bonds, W_i, w2p):
    def body(fb_ref, wi_ref, w2_ref, inp_ref, msg_ref, wm_ref):
        h = jnp.dot(fb_ref[...], wi_ref[...],
                    preferred_element_type=jnp.float32)
        inp_ref[...] = h
        m = jnp.maximum(h, 0.0)
        msg_ref[...] = m
        wm_ref[...] = m * w2_ref[...]

    return pl.pallas_call(
        body,
        grid=(NBP // BM,),
        in_specs=[pl.BlockSpec((BM, BOND_FDIM), lambda i: (i, 0)),
                  pl.BlockSpec((BOND_FDIM, H), lambda i: (0, 0)),
                  pl.BlockSpec((BM, 1), lambda i: (i, 0))],
        out_specs=[pl.BlockSpec((BM, H), lambda i: (i, 0)),
                   pl.BlockSpec((BM, H), lambda i: (i, 0)),
                   pl.BlockSpec((BM, H), lambda i: (i, 0))],
        out_shape=[jax.ShapeDtypeStruct((NBP, H), jnp.float32)] * 3,
    )(f_bonds, W_i, w2p)


def _update(g1, g2, inp, W_h, w2p):
    def body(g1_ref, g2_ref, inp_ref, wh_ref, w2_ref, msg_ref, wm_ref):
        w2 = w2_ref[...]
        pre = g1_ref[...] - g2_ref[...] * w2
        h = jnp.dot(pre, wh_ref[...], preferred_element_type=jnp.float32)
        m = jnp.maximum(inp_ref[...] + h, 0.0)
        msg_ref[...] = m
        wm_ref[...] = m * w2

    return pl.pallas_call(
        body,
        grid=(NBP // BM,),
        in_specs=[pl.BlockSpec((BM, H), lambda i: (i, 0)),
                  pl.BlockSpec((BM, H), lambda i: (i, 0)),
                  pl.BlockSpec((BM, H), lambda i: (i, 0)),
                  pl.BlockSpec((H, H), lambda i: (0, 0)),
                  pl.BlockSpec((BM, 1), lambda i: (i, 0))],
        out_specs=[pl.BlockSpec((BM, H), lambda i: (i, 0)),
                   pl.BlockSpec((BM, H), lambda i: (i, 0))],
        out_shape=[jax.ShapeDtypeStruct((NBP, H), jnp.float32)] * 2,
    )(g1, g2, inp, W_h, w2p)


def _readout(f_atoms_s, a_msg_s, Wo1, Wo2, b_o2):
    RB = 400

    def body(fa_ref, am_ref, w1_ref, w2_ref, b_ref, out_ref):
        h = (jnp.dot(fa_ref[...], w1_ref[...],
                     preferred_element_type=jnp.float32)
             + jnp.dot(am_ref[...], w2_ref[...],
                       preferred_element_type=jnp.float32)
             + b_ref[...])
        out_ref[...] = jnp.maximum(h, 0.0)

    return pl.pallas_call(
        body,
        grid=(N_MOLS * MOL_SIZE // RB,),
        in_specs=[pl.BlockSpec((RB, ATOM_FDIM), lambda i: (i, 0)),
                  pl.BlockSpec((RB, H), lambda i: (i, 0)),
                  pl.BlockSpec((ATOM_FDIM, H), lambda i: (0, 0)),
                  pl.BlockSpec((H, H), lambda i: (0, 0)),
                  pl.BlockSpec((1, H), lambda i: (0, 0))],
        out_specs=pl.BlockSpec((RB, H), lambda i: (i, 0)),
        out_shape=jax.ShapeDtypeStruct((N_MOLS * MOL_SIZE, H), jnp.float32),
    )(f_atoms_s, a_msg_s, Wo1, Wo2, b_o2)


def _pool(ah3, w3, dop3):
    def body(h_ref, w_ref, d_ref, out_ref):
        x = h_ref[...]
        w = w_ref[...]
        s = jnp.sum(x * w, axis=1)
        den = jnp.sum(w, axis=1)
        out_ref[...] = s * (d_ref[..., 0] / den)

    return pl.pallas_call(
        body,
        out_shape=jax.ShapeDtypeStruct((N_MOLS, H), jnp.float32),
    )(ah3, w3, dop3)


def kernel(f_atoms, f_bonds, w_atoms, w_bonds, a2b, b2a, b2revb,
           degree_of_polym, W_i, W_h, W_o, b_o):
    i32 = jnp.int32
    a2b_flat = jnp.pad(a2b.astype(i32).reshape(-1),
                       (0, NAP * MAX_NB - N_ATOMS * MAX_NB))
    b2a_p = jnp.pad(b2a.astype(i32), (0, NBG - N_BONDS))
    b2revb_p = jnp.pad(b2revb.astype(i32), (0, NBG - N_BONDS))
    w2p = jnp.pad(w_bonds, (0, NBP - N_BONDS))[:, None]

    inp, msg, wm = _stage0(f_bonds, W_i, w2p)
    for _ in range(DEPTH - 1):
        a_msg = _a_message(a2b_flat, wm)
        g1, g2 = _bond_gathers(b2a_p, b2revb_p, a_msg, msg)
        msg, wm = _update(g1, g2, inp, W_h, w2p)
    a_msg = _a_message(a2b_flat, wm)

    ah = _readout(f_atoms[1:N_ATOMS], a_msg[1:N_ATOMS],
                  W_o[:ATOM_FDIM], W_o[ATOM_FDIM:], b_o[None, :])
    mol_vecs = _pool(ah.reshape(N_MOLS, MOL_SIZE, H),
                     w_atoms[1:N_ATOMS].reshape(N_MOLS, MOL_SIZE, 1),
                     degree_of_polym[:, None, None])
    return mol_vecs


# speedup vs baseline: 1.5479x; 1.1162x over previous
import functools

import jax
import jax.numpy as jnp
from jax import lax
from jax.experimental import pallas as pl
from jax.experimental.pallas import tpu as pltpu
from jax.experimental.pallas import tpu_sc as plsc

N_ATOMS = 10001
N_BONDS = 320001
ATOM_FDIM = 128
BOND_FDIM = 144
H = 128
MAX_NB = 32
DEPTH = 3
N_MOLS = 100
MOL_SIZE = 100

NC, NS = 2, 16
NW = NC * NS
IDXB = 128
AB = 4
NBATCH_A = 80
NAP = NW * NBATCH_A * AB
NBATCH_B = 80
NBG = NW * NBATCH_B * IDXB
B0, B1 = 120, 40
BM = 1024
NBP = 313 * BM

_vmesh = plsc.VectorSubcoreMesh(core_axis_name="c", subcore_axis_name="s")


def _a_message(a2b_flat, wm):
    APT0 = B0 * AB
    ZR = 32
    WCH = 160

    @functools.partial(
        pl.kernel,
        out_type=jax.ShapeDtypeStruct((NAP, H), jnp.float32),
        mesh=_vmesh,
        scratch_types=[
            pltpu.VMEM((2, IDXB), jnp.int32),
            pltpu.VMEM((2, IDXB), jnp.int32),
            pltpu.VMEM((2, IDXB, H), jnp.float32),
            pltpu.VMEM((ZR, H), jnp.float32),
            pltpu.VMEM_SHARED((NS * APT0, H), jnp.float32),
            pltpu.SemaphoreType.DMA((2,)),
            pltpu.SemaphoreType.DMA((2,)),
            pltpu.SemaphoreType.DMA((2,)),
        ],
    )
    def k(a2b_hbm, wm_hbm, out_hbm, idx_v, dst_v, rows_v, zeros_v, acc_sh,
          sem_i, sem_g, sem_s):
        cid = lax.axis_index("c")
        sid = lax.axis_index("s")
        bstart = jnp.where(cid == 0, sid * B0, NS * B0 + sid * B1)
        npair = jnp.where(cid == 0, B0 // 2, B1 // 2)
        nzero = jnp.where(cid == 0, APT0 // ZR, (B1 * AB) // ZR)
        nwch = jnp.where(cid == 0, APT0 // WCH, (B1 * AB) // WCH)
        tibase = bstart * IDXB

        for i in range(ZR):
            for j in range(H // 16):
                zeros_v[i, pl.ds(j * 16, 16)] = jnp.zeros((16,), jnp.float32)

        @pl.loop(0, nzero)
        def _(z):
            pltpu.sync_copy(zeros_v,
                            acc_sh.at[pl.ds(sid * APT0 + z * ZR, ZR)])

        for p in range(2):
            off = sid * APT0 + p * AB
            dvp = dst_v.at[p]
            for j in range(IDXB // 16):
                dvp[pl.ds(j * 16, 16)] = (
                    jnp.full((16,), j // 2, jnp.int32) + off)

        def issue_idx(b, p):
            pltpu.async_copy(a2b_hbm.at[pl.ds(tibase + b * IDXB, IDXB)],
                             idx_v.at[p], sem_i.at[p])

        issue_idx(0, 0)
        issue_idx(1, 1)

        @pl.loop(0, npair)
        def _(g):
            b0 = 2 * g
            gathers = []
            for p in range(2):
                pltpu.make_async_copy(
                    a2b_hbm.at[pl.ds(tibase, IDXB)], idx_v.at[p],
                    sem_i.at[p]).wait()
                gathers.append(pltpu.async_copy(
                    wm_hbm.at[idx_v.at[p]], rows_v.at[p], sem_g.at[p]))
            for c in gathers:
                c.wait()
            adds = []
            for p in range(2):
                adds.append(pltpu.async_copy(
                    rows_v.at[p], acc_sh.at[dst_v.at[p]], sem_s.at[p],
                    add=True))

            @pl.when(g < npair - 1)
            def _():
                issue_idx(b0 + 2, 0)
                issue_idx(b0 + 3, 1)

            for c in adds:
                c.wait()
            for p in range(2):
                dvp = dst_v.at[p]
                for j in range(IDXB // 16):
                    sl = pl.ds(j * 16, 16)
                    dvp[sl] = dvp[sl] + (2 * AB)

        @pl.loop(0, nwch)
        def _(c):
            pltpu.sync_copy(
                acc_sh.at[pl.ds(sid * APT0 + c * WCH, WCH)],
                out_hbm.at[pl.ds(bstart * AB + c * WCH, WCH)])

    return k(a2b_flat, wm)


def _bond_gathers(b2a_p, b2revb_p, a_msg, msg):
    @functools.partial(
        pl.kernel,
        out_type=[jax.ShapeDtypeStruct((NBG, H), jnp.float32),
                  jax.ShapeDtypeStruct((NBG, H), jnp.float32)],
        mesh=_vmesh,
        scratch_types=[
            pltpu.VMEM((2, IDXB), jnp.int32),
            pltpu.VMEM((2, IDXB), jnp.int32),
            pltpu.VMEM((2, IDXB, H), jnp.float32),
            pltpu.VMEM((2, IDXB, H), jnp.float32),
            pltpu.SemaphoreType.DMA((2,)),
            pltpu.SemaphoreType.DMA((2,)),
            pltpu.SemaphoreType.DMA((2,)),
        ],
    )
    def k(b2a_hbm, b2revb_hbm, am_hbm, msg_hbm, g1_hbm, g2_hbm,
          idx1_v, idx2_v, r1_v, r2_v, sem_i, sem_g, sem_w):
        cid = lax.axis_index("c")
        sid = lax.axis_index("s")
        bstart = jnp.where(cid == 0, sid * B0, NS * B0 + sid * B1)
        npair = jnp.where(cid == 0, B0 // 2, B1 // 2)
        tbase = bstart * IDXB

        def issue_idx(b, p):
            base = tbase + b * IDXB
            pltpu.async_copy(b2a_hbm.at[pl.ds(base, IDXB)], idx1_v.at[p],
                             sem_i.at[p])
            pltpu.async_copy(b2revb_hbm.at[pl.ds(base, IDXB)], idx2_v.at[p],
                             sem_i.at[p])

        issue_idx(0, 0)
        issue_idx(1, 1)

        @pl.loop(0, npair)
        def _(g):
            b0 = 2 * g

            @pl.when(g > 0)
            def _():
                for p in range(2):
                    pltpu.make_async_copy(r1_v.at[p],
                                          g1_hbm.at[pl.ds(0, IDXB)],
                                          sem_w.at[p]).wait()
                    pltpu.make_async_copy(r2_v.at[p],
                                          g2_hbm.at[pl.ds(0, IDXB)],
                                          sem_w.at[p]).wait()

            gathers = []
            for p in range(2):
                pltpu.make_async_copy(b2a_hbm.at[pl.ds(tbase, IDXB)],
                                      idx1_v.at[p], sem_i.at[p]).wait()
                pltpu.make_async_copy(b2revb_hbm.at[pl.ds(tbase, IDXB)],
                                      idx2_v.at[p], sem_i.at[p]).wait()
                gathers.append(pltpu.async_copy(
                    am_hbm.at[idx1_v.at[p]], r1_v.at[p], sem_g.at[p]))
                gathers.append(pltpu.async_copy(
                    msg_hbm.at[idx2_v.at[p]], r2_v.at[p], sem_g.at[p]))
            for c in gathers:
                c.wait()
            for p in range(2):
                base = tbase + (b0 + p) * IDXB
                pltpu.async_copy(r1_v.at[p], g1_hbm.at[pl.ds(base, IDXB)],
                                 sem_w.at[p])
                pltpu.async_copy(r2_v.at[p], g2_hbm.at[pl.ds(base, IDXB)],
                                 sem_w.at[p])

            @pl.when(g < npair - 1)
            def _():
                issue_idx(b0 + 2, 0)
                issue_idx(b0 + 3, 1)

        for p in range(2):
            pltpu.make_async_copy(r1_v.at[p], g1_hbm.at[pl.ds(0, IDXB)],
                                  sem_w.at[p]).wait()
            pltpu.make_async_copy(r2_v.at[p], g2_hbm.at[pl.ds(0, IDXB)],
                                  sem_w.at[p]).wait()

    return k(b2a_p, b2revb_p, a_msg, msg)


def _stage0(f_bonds, W_i, w2p):
    def body(fb_ref, wi_ref, w2_ref, inp_ref, msg_ref, wm_ref):
        h = jnp.dot(fb_ref[...], wi_ref[...],
                    preferred_element_type=jnp.float32)
        inp_ref[...] = h
        m = jnp.maximum(h, 0.0)
        msg_ref[...] = m
        wm_ref[...] = m * w2_ref[...]

    return pl.pallas_call(
        body,
        grid=(NBP // BM,),
        in_specs=[pl.BlockSpec((BM, BOND_FDIM), lambda i: (i, 0)),
                  pl.BlockSpec((BOND_FDIM, H), lambda i: (0, 0)),
                  pl.BlockSpec((BM, 1), lambda i: (i, 0))],
        out_specs=[pl.BlockSpec((BM, H), lambda i: (i, 0)),
                   pl.BlockSpec((BM, H), lambda i: (i, 0)),
                   pl.BlockSpec((BM, H), lambda i: (i, 0))],
        out_shape=[jax.ShapeDtypeStruct((NBP, H), jnp.float32)] * 3,
    )(f_bonds, W_i, w2p)


def _update(g1, g2, inp, W_h, w2p):
    def body(g1_ref, g2_ref, inp_ref, wh_ref, w2_ref, msg_ref, wm_ref):
        w2 = w2_ref[...]
        pre = g1_ref[...] - g2_ref[...] * w2
        h = jnp.dot(pre, wh_ref[...], preferred_element_type=jnp.float32)
        m = jnp.maximum(inp_ref[...] + h, 0.0)
        msg_ref[...] = m
        wm_ref[...] = m * w2

    return pl.pallas_call(
        body,
        grid=(NBP // BM,),
        in_specs=[pl.BlockSpec((BM, H), lambda i: (i, 0)),
                  pl.BlockSpec((BM, H), lambda i: (i, 0)),
                  pl.BlockSpec((BM, H), lambda i: (i, 0)),
                  pl.BlockSpec((H, H), lambda i: (0, 0)),
                  pl.BlockSpec((BM, 1), lambda i: (i, 0))],
        out_specs=[pl.BlockSpec((BM, H), lambda i: (i, 0)),
                   pl.BlockSpec((BM, H), lambda i: (i, 0))],
        out_shape=[jax.ShapeDtypeStruct((NBP, H), jnp.float32)] * 2,
    )(g1, g2, inp, W_h, w2p)


def _readout(f_atoms_s, a_msg_s, Wo1, Wo2, b_o2):
    RB = 400

    def body(fa_ref, am_ref, w1_ref, w2_ref, b_ref, out_ref):
        h = (jnp.dot(fa_ref[...], w1_ref[...],
                     preferred_element_type=jnp.float32)
             + jnp.dot(am_ref[...], w2_ref[...],
                       preferred_element_type=jnp.float32)
             + b_ref[...])
        out_ref[...] = jnp.maximum(h, 0.0)

    return pl.pallas_call(
        body,
        grid=(N_MOLS * MOL_SIZE // RB,),
        in_specs=[pl.BlockSpec((RB, ATOM_FDIM), lambda i: (i, 0)),
                  pl.BlockSpec((RB, H), lambda i: (i, 0)),
                  pl.BlockSpec((ATOM_FDIM, H), lambda i: (0, 0)),
                  pl.BlockSpec((H, H), lambda i: (0, 0)),
                  pl.BlockSpec((1, H), lambda i: (0, 0))],
        out_specs=pl.BlockSpec((RB, H), lambda i: (i, 0)),
        out_shape=jax.ShapeDtypeStruct((N_MOLS * MOL_SIZE, H), jnp.float32),
    )(f_atoms_s, a_msg_s, Wo1, Wo2, b_o2)


def _pool(ah3, w3, dop3):
    def body(h_ref, w_ref, d_ref, out_ref):
        x = h_ref[...]
        w = w_ref[...]
        s = jnp.sum(x * w, axis=1)
        den = jnp.sum(w, axis=1)
        out_ref[...] = s * (d_ref[..., 0] / den)

    return pl.pallas_call(
        body,
        out_shape=jax.ShapeDtypeStruct((N_MOLS, H), jnp.float32),
    )(ah3, w3, dop3)


def kernel(f_atoms, f_bonds, w_atoms, w_bonds, a2b, b2a, b2revb,
           degree_of_polym, W_i, W_h, W_o, b_o):
    i32 = jnp.int32
    a2b_flat = jnp.pad(a2b.astype(i32).reshape(-1),
                       (0, NAP * MAX_NB - N_ATOMS * MAX_NB))
    b2a_p = jnp.pad(b2a.astype(i32), (0, NBG - N_BONDS))
    b2revb_p = jnp.pad(b2revb.astype(i32), (0, NBG - N_BONDS))
    w2p = jnp.pad(w_bonds, (0, NBP - N_BONDS))[:, None]

    inp, msg, wm = _stage0(f_bonds, W_i, w2p)
    for _ in range(DEPTH - 1):
        a_msg = _a_message(a2b_flat, wm)
        g1, g2 = _bond_gathers(b2a_p, b2revb_p, a_msg, msg)
        msg, wm = _update(g1, g2, inp, W_h, w2p)
    a_msg = _a_message(a2b_flat, wm)

    ah = _readout(f_atoms[1:N_ATOMS], a_msg[1:N_ATOMS],
                  W_o[:ATOM_FDIM], W_o[ATOM_FDIM:], b_o[None, :])
    mol_vecs = _pool(ah.reshape(N_MOLS, MOL_SIZE, H),
                     w_atoms[1:N_ATOMS].reshape(N_MOLS, MOL_SIZE, 1),
                     degree_of_polym[:, None, None])
    return mol_vecs
